# baseline (speedup 1.0000x reference)
import functools

import jax
import jax.numpy as jnp
import numpy as np
from jax.experimental import pallas as pl

T = 1000
TPAD = 1024
B = 128
L = 3 * 224 * 224
ROWS = 8


def _combine_body(ts_ref, beta_ref, x_ref, n_ref, out_ref):
    la = jnp.log1p(-beta_ref[0, :])
    i = jax.lax.broadcasted_iota(jnp.int32, (ROWS, TPAD), 1)
    mask = i <= ts_ref[...]
    s = jnp.sum(jnp.where(mask, la[None, :], 0.0), axis=1, keepdims=True)
    cp = jnp.exp(s)
    sa = jnp.sqrt(cp)
    sb = jnp.sqrt(1.0 - cp)
    out_ref[...] = sa * x_ref[...] + sb * n_ref[...]


@jax.jit
def kernel(x, timestep, beta_schedule):
    noise = jax.random.normal(jax.random.key(42), x.shape, dtype=x.dtype)
    x2 = x.reshape(B, L)
    n2 = noise.reshape(B, L)
    ts = timestep.reshape(B, 1)
    beta = jnp.pad(beta_schedule, (0, TPAD - T)).reshape(1, TPAD)
    grid = (B // ROWS,)
    out = pl.pallas_call(
        _combine_body,
        grid=grid,
        in_specs=[
            pl.BlockSpec((ROWS, 1), lambda b: (b, 0)),
            pl.BlockSpec((1, TPAD), lambda b: (0, 0)),
            pl.BlockSpec((ROWS, L), lambda b: (b, 0)),
            pl.BlockSpec((ROWS, L), lambda b: (b, 0)),
        ],
        out_specs=pl.BlockSpec((ROWS, L), lambda b: (b, 0)),
        out_shape=jax.ShapeDtypeStruct((B, L), x.dtype),
    )(ts, beta, x2, n2)
    return out.reshape(x.shape), noise


# speedup vs baseline: 1.9495x; 1.9495x over previous
import jax
import jax.numpy as jnp
import numpy as np
from jax.experimental import pallas as pl
from jax.experimental.pallas import tpu as pltpu

T = 1000
TPAD = 1024
B = 128
R = 1176
C = 128
L = R * C
BB = 8

_K1 = np.uint32(42)
_K2 = np.uint32(0x1BD11BDA ^ 42)
_LO = np.float32(np.nextafter(np.float32(-1.0), np.float32(0.0)))

_CC = (np.float32(2.122917214274262), np.float32(-0.34995386083658697),
       np.float32(-0.004681780622241893), np.float32(0.0021330589779123277))
_CT = (np.float32(4.005365305566973), np.float32(1.4192557312029732),
       np.float32(0.032923790098936645))


def _sched_body(ts_ref, beta_ref, sa_ref, sb_ref):
    la = jnp.log1p(-beta_ref[0, :])
    i = jax.lax.broadcasted_iota(jnp.int32, (B, TPAD), 1)
    mask = i <= ts_ref[...]
    s = jnp.sum(jnp.where(mask, la[None, :], 0.0), axis=1, keepdims=True)
    cp = jnp.exp(s)
    sa_ref[...] = jnp.sqrt(cp)
    sb_ref[...] = jnp.sqrt(1.0 - cp)


def _rotl(v, r):
    return (v << np.uint32(r)) | (v >> np.uint32(32 - r))


def _main_body(sa_ref, sb_ref, x_ref, out_ref, noise_ref):
    pid = pl.program_id(0)
    ir = jax.lax.broadcasted_iota(jnp.uint32, (R, C), 0)
    ic = jax.lax.broadcasted_iota(jnp.uint32, (R, C), 1)
    jbase = ir * np.uint32(C) + ic
    for bi in range(BB):
        b = pid * BB + bi
        base = (b * L + 42).astype(jnp.uint32)
        x1 = jbase + base
        x0 = x1
        x1 = _rotl(x1, 13) ^ x0
        for r in (15, 26, 6):
            x0 = x0 + x1
            x1 = _rotl(x1, r) ^ x0
        x0 = x0 + _K1
        x1 = x1 + np.uint32(_K2 + 1)
        for r in (17, 29, 16, 24):
            x0 = x0 + x1
            x1 = _rotl(x1, r) ^ x0
        x0 = x0 + _K2
        x1 = x1 + np.uint32(2)
        for r in (13, 15, 26, 6):
            x0 = x0 + x1
            x1 = _rotl(x1, r) ^ x0
        x1 = x1 + np.uint32(_K1 + 3)
        for r in (17, 29, 16, 24):
            x0 = x0 + x1
            x1 = _rotl(x1, r) ^ x0
        x0 = x0 + _K1
        x1 = x1 + np.uint32(_K2 + 4)
        for r in (13, 15, 26, 6):
            x0 = x0 + x1
            x1 = _rotl(x1, r) ^ x0
        x0 = x0 + _K2
        x1 = x1 + np.uint32(5)
        bits = x0 ^ x1

        g = jax.lax.bitcast_convert_type((bits >> np.uint32(9))
                                         | np.uint32(0x40000000), jnp.float32)
        u = jnp.maximum(g - np.float32(3.0), _LO)
        y = jnp.log(1.0 - u * u)
        q = y + np.float32(2.5)
        pc = ((_CC[3] * q + _CC[2]) * q + _CC[1]) * q + _CC[0]
        st = jnp.sqrt(-y) - np.float32(3.0)
        pt = (_CT[2] * st + _CT[1]) * st + _CT[0]
        p = jnp.where(y > np.float32(-5.0), pc, pt)
        z = u * p

        noise_ref[bi] = z
        sa = sa_ref[b, 0]
        sb = sb_ref[b, 0]
        out_ref[bi] = sa * x_ref[bi] + sb * z


@jax.jit
def kernel(x, timestep, beta_schedule):
    ts = timestep.reshape(B, 1)
    beta = jnp.pad(beta_schedule, (0, TPAD - T)).reshape(1, TPAD)
    sa, sb = pl.pallas_call(
        _sched_body,
        in_specs=[
            pl.BlockSpec((B, 1), lambda: (0, 0)),
            pl.BlockSpec((1, TPAD), lambda: (0, 0)),
        ],
        out_specs=[
            pl.BlockSpec((B, 1), lambda: (0, 0)),
            pl.BlockSpec((B, 1), lambda: (0, 0)),
        ],
        out_shape=[
            jax.ShapeDtypeStruct((B, 1), jnp.float32),
            jax.ShapeDtypeStruct((B, 1), jnp.float32),
        ],
    )(ts, beta)

    x3 = x.reshape(B, R, C)
    out, noise = pl.pallas_call(
        _main_body,
        grid=(B // BB,),
        in_specs=[
            pl.BlockSpec(memory_space=pltpu.SMEM),
            pl.BlockSpec(memory_space=pltpu.SMEM),
            pl.BlockSpec((BB, R, C), lambda i: (i, 0, 0)),
        ],
        out_specs=[
            pl.BlockSpec((BB, R, C), lambda i: (i, 0, 0)),
            pl.BlockSpec((BB, R, C), lambda i: (i, 0, 0)),
        ],
        out_shape=[
            jax.ShapeDtypeStruct((B, R, C), x.dtype),
            jax.ShapeDtypeStruct((B, R, C), x.dtype),
        ],
    )(sa, sb, x3)
    return out.reshape(x.shape), noise.reshape(x.shape)
